# Initial kernel scaffold; baseline (speedup 1.0000x reference)
import jax
import jax.numpy as jnp
from jax.experimental import pallas as pl


def kernel(input_ids, weight):
    raise NotImplementedError("write your pallas kernel here")


# speedup vs baseline: 1.0937x; 1.0937x over previous
import functools

import jax
import jax.numpy as jnp
from jax import lax
from jax.experimental import pallas as pl
from jax.experimental.pallas import tpu as pltpu
from jax.experimental.pallas import tpu_sc as plsc

NC = 2
NS = 16
NW = NC * NS


def _gather_call(n_total, n_per_w, ch, d):
    n_ch = n_per_w // ch
    mesh = plsc.VectorSubcoreMesh(core_axis_name="c", subcore_axis_name="s")

    @functools.partial(
        pl.kernel,
        mesh=mesh,
        out_type=jax.ShapeDtypeStruct((n_total, d), jnp.float32),
        scratch_types=[
            pltpu.VMEM((ch,), jnp.int32),
            pltpu.VMEM((ch, d), jnp.float32),
            pltpu.SemaphoreType.DMA,
        ],
        compiler_params=pltpu.CompilerParams(use_tc_tiling_on_sc=False),
    )
    def gather_kernel(idx_hbm, table_hbm, out_hbm, idx_v, rows_v, sem):
        wid = lax.axis_index("s") * NC + lax.axis_index("c")
        base = wid * n_per_w

        def body(i, carry):
            off = base + i * ch
            pltpu.sync_copy(idx_hbm.at[pl.ds(off, ch)], idx_v)
            pltpu.async_copy(table_hbm.at[idx_v], rows_v, sem).wait()
            pltpu.sync_copy(rows_v, out_hbm.at[pl.ds(off, ch)])
            return carry

        lax.fori_loop(0, n_ch, body, 0)

    return gather_kernel


def kernel(input_ids, weight):
    b, h = input_ids.shape
    v, d = weight.shape
    n_total = b * h
    n_per_w = n_total // NW
    ch = 1024

    flat_ids = input_ids.reshape(n_total)
    out = _gather_call(n_total, n_per_w, ch, d)(flat_ids, weight)
    return out.reshape(b, h, d)


# speedup vs baseline: 1.1123x; 1.0170x over previous
import functools

import jax
import jax.numpy as jnp
from jax import lax
from jax.experimental import pallas as pl
from jax.experimental.pallas import tpu as pltpu
from jax.experimental.pallas import tpu_sc as plsc

NC = 2
NS = 16
NW = NC * NS
NBUF = 8
LAG = 4


def _gather_call(n_total, n_per_w, ch, d):
    n_ch = n_per_w // ch
    n_rounds = n_ch // NBUF
    assert n_ch % NBUF == 0 and n_rounds >= 2
    mesh = plsc.VectorSubcoreMesh(core_axis_name="c", subcore_axis_name="s")

    @functools.partial(
        pl.kernel,
        mesh=mesh,
        out_type=jax.ShapeDtypeStruct((n_total, d), jnp.float32),
        scratch_types=[
            pltpu.VMEM((n_per_w,), jnp.int32),
            pltpu.VMEM((NBUF, ch, d), jnp.float32),
            pltpu.SemaphoreType.DMA((NBUF,)),
            pltpu.SemaphoreType.DMA((NBUF,)),
        ],
        compiler_params=pltpu.CompilerParams(use_tc_tiling_on_sc=False),
    )
    def gather_kernel(idx_hbm, table_hbm, out_hbm, idx_v, rows_v, semg, semw):
        wid = lax.axis_index("s") * NC + lax.axis_index("c")
        base = wid * n_per_w
        pltpu.sync_copy(idx_hbm.at[pl.ds(base, n_per_w)], idx_v)

        def start_gather(j, b):
            pltpu.async_copy(
                table_hbm.at[idx_v.at[pl.ds(j * ch, ch)]],
                rows_v.at[b], semg.at[b])

        def wait_gather(j, b):
            pltpu.make_async_copy(
                table_hbm.at[idx_v.at[pl.ds(j * ch, ch)]],
                rows_v.at[b], semg.at[b]).wait()

        def start_wb(j, b):
            pltpu.async_copy(
                rows_v.at[b], out_hbm.at[pl.ds(base + j * ch, ch)], semw.at[b])

        def wait_wb(j, b):
            pltpu.make_async_copy(
                rows_v.at[b], out_hbm.at[pl.ds(base + j * ch, ch)],
                semw.at[b]).wait()

        for b in range(LAG):
            start_gather(b, b)

        for b in range(NBUF):
            jg = b + LAG
            if jg < NBUF:
                start_gather(jg, jg)
            else:
                wait_wb(jg - NBUF, jg % NBUF)
                start_gather(jg, jg % NBUF)
            wait_gather(b, b)
            start_wb(b, b)

        def round_body(r, carry):
            g = r * NBUF
            for b in range(NBUF):
                j = g + b
                jg = j + LAG
                bg = (b + LAG) % NBUF
                wait_wb(jg - NBUF, bg)
                start_gather(jg, bg)
                wait_gather(j, b)
                start_wb(j, b)
            return carry

        lax.fori_loop(1, n_rounds - 1, round_body, 0)

        g = (n_rounds - 1) * NBUF
        for b in range(NBUF):
            j = g + b
            jg = j + LAG
            if jg < n_ch:
                bg = (b + LAG) % NBUF
                wait_wb(jg - NBUF, bg)
                start_gather(jg, bg)
            wait_gather(j, b)
            start_wb(j, b)

        for b in range(NBUF):
            wait_wb(g + b, b)

    return gather_kernel


def kernel(input_ids, weight):
    b, h = input_ids.shape
    v, d = weight.shape
    n_total = b * h
    n_per_w = n_total // NW
    ch = 400

    flat_ids = input_ids.reshape(n_total)
    out = _gather_call(n_total, n_per_w, ch, d)(flat_ids, weight)
    return out.reshape(b, h, d)
